# baseline (speedup 1.0000x reference)
import functools
import math

import jax
import jax.numpy as jnp
from jax.experimental import pallas as pl
from jax.experimental.pallas import tpu as pltpu

_VMEM_LIMIT = 64 * 1024 * 1024


def _largest_divisor_leq(n, k):
    k = max(1, min(n, k))
    while n % k:
        k -= 1
    return k


def _dot(a, b):
    return jax.lax.dot(a, b, preferred_element_type=jnp.float32)


def _se_kernel(x_ref, s_ref, st_ref, t_ref, tt_ref, e_ref, et_ref, m_ref,
               w1t_ref, w2t_ref, o_ref, *, inv_hw):
    x = x_ref[...]
    m = m_ref[...]
    p = _dot(x, s_ref[...])
    y = _dot(e_ref[...], _dot(p, t_ref[...]) * m) * inv_hw
    z1 = jnp.maximum(_dot(y, w1t_ref[...]), 0.0)
    gate = jax.nn.sigmoid(_dot(z1, w2t_ref[...]))
    gate_g = _dot(_dot(et_ref[...], gate) * m, tt_ref[...])
    o_ref[...] = x * _dot(gate_g, st_ref[...])


def kernel(x_nchw, fc1_w_t, fc2_w):
    B, C, H, W = x_nchw.shape
    C1, Cr = fc1_w_t.shape
    assert C1 == C and fc2_w.shape == (C, Cr)
    HW = H * W
    inv_hw = 1.0 / HW

    grp = 128 // math.gcd(HW, 128)
    assert C % grp == 0
    L = grp * HW
    R = C // grp
    x_flat = x_nchw.reshape(B * R, L)

    itemsize = x_nchw.dtype.itemsize
    per_batch_bytes = C * HW * itemsize
    tb = _largest_divisor_leq(B, max(1, (4 << 20) // per_batch_bytes))
    TR = tb * R

    f32 = jnp.float32
    iota = jax.lax.broadcasted_iota
    seg = (iota(jnp.int32, (L, grp), 0) // HW ==
           iota(jnp.int32, (L, grp), 1)).astype(f32)
    seg_t = seg.T
    tmat = (iota(jnp.int32, (grp, C), 1) % grp ==
            iota(jnp.int32, (grp, C), 0)).astype(f32)
    tmat_t = tmat.T
    emat = (iota(jnp.int32, (tb, TR), 1) // R ==
            iota(jnp.int32, (tb, TR), 0)).astype(f32)
    emat_t = emat.T
    mmat = (iota(jnp.int32, (TR, C), 0) % R ==
            iota(jnp.int32, (TR, C), 1) // grp).astype(f32)
    w2t = fc2_w.T

    cost = pl.CostEstimate(
        flops=B * (4 * C * HW * grp + 4 * C * Cr),
        transcendentals=B * C,
        bytes_accessed=2 * B * C * HW * itemsize + 2 * L * grp * 4,
    )
    out = pl.pallas_call(
        functools.partial(_se_kernel, inv_hw=inv_hw),
        out_shape=jax.ShapeDtypeStruct((B * R, L), x_flat.dtype),
        grid=(B // tb,),
        in_specs=[
            pl.BlockSpec((TR, L), lambda b: (b, 0)),
            pl.BlockSpec((L, grp), lambda b: (0, 0)),
            pl.BlockSpec((grp, L), lambda b: (0, 0)),
            pl.BlockSpec((grp, C), lambda b: (0, 0)),
            pl.BlockSpec((C, grp), lambda b: (0, 0)),
            pl.BlockSpec((tb, TR), lambda b: (0, 0)),
            pl.BlockSpec((TR, tb), lambda b: (0, 0)),
            pl.BlockSpec((TR, C), lambda b: (0, 0)),
            pl.BlockSpec((C, Cr), lambda b: (0, 0)),
            pl.BlockSpec((Cr, C), lambda b: (0, 0)),
        ],
        out_specs=pl.BlockSpec((TR, L), lambda b: (b, 0)),
        compiler_params=pltpu.CompilerParams(
            dimension_semantics=("parallel",),
            vmem_limit_bytes=_VMEM_LIMIT),
        cost_estimate=cost,
    )(x_flat, seg, seg_t, tmat, tmat_t, emat, emat_t, mmat, fc1_w_t, w2t)
    return out.reshape(B, C, H, W)


# speedup vs baseline: 5.2510x; 5.2510x over previous
import functools

import jax
import jax.numpy as jnp
from jax.experimental import pallas as pl
from jax.experimental.pallas import tpu as pltpu

_VMEM_LIMIT = 64 * 1024 * 1024


def _largest_divisor_leq(n, k):
    k = max(1, min(n, k))
    while n % k:
        k -= 1
    return k


def _se_kernel(x_ref, w1t_ref, w2_ref, o_ref, *, inv_hw):
    x = x_ref[...]
    y = jnp.sum(x, axis=-1, keepdims=True) * inv_hw
    z1 = jnp.maximum(jnp.sum(w1t_ref[...] * y, axis=1, keepdims=True), 0.0)
    z2 = jnp.sum(w2_ref[...] * z1, axis=-1, keepdims=True)
    o_ref[...] = x * jax.nn.sigmoid(z2)


def kernel(x_nchw, fc1_w_t, fc2_w):
    B, C, H, W = x_nchw.shape
    C1, Cr = fc1_w_t.shape
    assert C1 == C and fc2_w.shape == (C, Cr)
    HW = H * W
    x = x_nchw.reshape(B, C, HW)

    itemsize = x_nchw.dtype.itemsize
    per_batch_bytes = C * HW * itemsize
    tb = _largest_divisor_leq(B, max(1, (6 << 20) // per_batch_bytes))

    out = pl.pallas_call(
        functools.partial(_se_kernel, inv_hw=1.0 / HW),
        out_shape=jax.ShapeDtypeStruct((B, C, HW), x.dtype),
        grid=(B // tb,),
        in_specs=[
            pl.BlockSpec((tb, C, HW), lambda b: (b, 0, 0)),
            pl.BlockSpec((C, Cr), lambda b: (0, 0)),
            pl.BlockSpec((C, Cr), lambda b: (0, 0)),
        ],
        out_specs=pl.BlockSpec((tb, C, HW), lambda b: (b, 0, 0)),
        compiler_params=pltpu.CompilerParams(
            dimension_semantics=("parallel",),
            vmem_limit_bytes=_VMEM_LIMIT),
    )(x, fc1_w_t, fc2_w)
    return out.reshape(B, C, H, W)


# speedup vs baseline: 5.2871x; 1.0069x over previous
import functools

import jax
import jax.numpy as jnp
from jax.experimental import pallas as pl
from jax.experimental.pallas import tpu as pltpu

_VMEM_LIMIT = 64 * 1024 * 1024


def _largest_divisor_leq(n, k):
    k = max(1, min(n, k))
    while n % k:
        k -= 1
    return k


def _se_kernel(x_ref, w1t_ref, w2_ref, o_ref, *, inv_hw):
    x = x_ref[...]
    y = jnp.sum(x, axis=-1, keepdims=True) * inv_hw
    z1 = jnp.maximum(jnp.sum(w1t_ref[...] * y, axis=1, keepdims=True), 0.0)
    z2 = jnp.sum(w2_ref[...] * z1, axis=-1, keepdims=True)
    o_ref[...] = x * jax.nn.sigmoid(z2)


def kernel(x_nchw, fc1_w_t, fc2_w):
    B, C, H, W = x_nchw.shape
    C1, Cr = fc1_w_t.shape
    assert C1 == C and fc2_w.shape == (C, Cr)
    HW = H * W
    x = x_nchw.reshape(B, C, HW)

    itemsize = x_nchw.dtype.itemsize
    per_batch_bytes = C * HW * itemsize
    tb = _largest_divisor_leq(B, max(1, (8 << 20) // per_batch_bytes))

    out = pl.pallas_call(
        functools.partial(_se_kernel, inv_hw=1.0 / HW),
        out_shape=jax.ShapeDtypeStruct((B, C, HW), x.dtype),
        grid=(B // tb,),
        in_specs=[
            pl.BlockSpec((tb, C, HW), lambda b: (b, 0, 0)),
            pl.BlockSpec((C, Cr), lambda b: (0, 0)),
            pl.BlockSpec((C, Cr), lambda b: (0, 0)),
        ],
        out_specs=pl.BlockSpec((tb, C, HW), lambda b: (b, 0, 0)),
        compiler_params=pltpu.CompilerParams(
            dimension_semantics=("parallel",),
            vmem_limit_bytes=_VMEM_LIMIT),
    )(x, fc1_w_t, fc2_w)
    return out.reshape(B, C, H, W)
